# baseline (speedup 1.0000x reference)
import functools
import math

import jax
import jax.numpy as jnp
from jax import lax
from jax.experimental import pallas as pl
from jax.experimental.pallas import tpu as pltpu
from jax.experimental.pallas import tpu_sc as plsc

B, N, CIN, COUT, K = 16, 4096, 128, 128, 16
NSAMP = 1024
BN_EPS = 1e-5


def _mlp_body(x_ref, w_ref, b_ref, gamma_ref, beta_ref, mean_ref, var_ref,
              o_ref):
    acc = jnp.dot(x_ref[...], w_ref[...], preferred_element_type=jnp.float32)
    scale = gamma_ref[...] / jnp.sqrt(var_ref[...] + BN_EPS)
    o_ref[...] = jnp.maximum(
        (acc + b_ref[...] - mean_ref[...]) * scale + beta_ref[...], 0.0)


def _mlp(xf, W, b, gamma, beta, mean, var):
    M = B * N
    BM = 2048
    vec = lambda v: v.reshape(1, COUT)
    return pl.pallas_call(
        _mlp_body,
        grid=(M // BM,),
        in_specs=[
            pl.BlockSpec((BM, CIN), lambda i: (i, 0)),
            pl.BlockSpec((CIN, COUT), lambda i: (0, 0)),
            pl.BlockSpec((1, COUT), lambda i: (0, 0)),
            pl.BlockSpec((1, COUT), lambda i: (0, 0)),
            pl.BlockSpec((1, COUT), lambda i: (0, 0)),
            pl.BlockSpec((1, COUT), lambda i: (0, 0)),
            pl.BlockSpec((1, COUT), lambda i: (0, 0)),
        ],
        out_specs=pl.BlockSpec((BM, COUT), lambda i: (i, 0)),
        out_shape=jax.ShapeDtypeStruct((M, COUT), jnp.float32),
    )(xf, W, vec(b), vec(gamma), vec(beta), vec(mean), vec(var))


_BLK = 128
_NBLK = N // _BLK
_FLUSH = 128


_HB = B // 2


def _fps_body(s0, nsteps, px_ref, py_ref, pz_ref, cin_ref, din_ref, fid_ref,
              pox_ref, poy_ref, poz_ref, cout_ref, dout_ref, sfid_ref,
              sx_ref, sy_ref, sz_ref):
    lane_blk = lax.broadcasted_iota(jnp.int32, (_HB, _BLK), 1)
    slane = lax.broadcasted_iota(jnp.int32, (_HB, _FLUSH), 1)
    neg_inf = jnp.float32(-jnp.inf)
    big = jnp.int32(N)
    boffs = [
        (lax.broadcasted_iota(jnp.int32, (_HB, 1), 0) + h * _HB) * N
        for h in range(2)
    ]
    rows = [pl.ds(h * _HB, _HB) for h in range(2)]
    dist_ref = dout_ref

    carry0 = []
    if s0 == 0:
        for h in range(2):
            r = rows[h]
            cx = px_ref[r, 0:1]
            cy = py_ref[r, 0:1]
            cz = pz_ref[r, 0:1]
            dx = px_ref[r, :] - cx
            dy = py_ref[r, :] - cy
            dz = pz_ref[r, :] - cz
            dist_ref[r, :] = dx * dx + dy * dy + dz * dz
            sfid_ref[r, :] = jnp.where(slane == 0, boffs[h], 0)
            sx_ref[r, :] = jnp.where(slane == 0, cx, 0.0)
            sy_ref[r, :] = jnp.where(slane == 0, cy, 0.0)
            sz_ref[r, :] = jnp.where(slane == 0, cz, 0.0)
            carry0 += [cx, cy, cz]
    else:
        dist_ref[...] = din_ref[...]
        for h in range(2):
            r = rows[h]
            carry0 += [cin_ref[r, 0:1], cin_ref[r, 1:2], cin_ref[r, 2:3]]

    def halfstep(h, i, cx, cy, cz):
        r = rows[h]
        macc = jnp.full((_HB, _BLK), neg_inf, jnp.float32)
        bacc = jnp.zeros((_HB, _BLK), jnp.int32)
        xacc = jnp.zeros((_HB, _BLK), jnp.float32)
        yacc = jnp.zeros((_HB, _BLK), jnp.float32)
        zacc = jnp.zeros((_HB, _BLK), jnp.float32)
        for blk in range(_NBLK):
            sl = pl.ds(blk * _BLK, _BLK)
            pxb = px_ref[r, sl]
            pyb = py_ref[r, sl]
            pzb = pz_ref[r, sl]
            dxb = pxb - cx
            dyb = pyb - cy
            dzb = pzb - cz
            db = dxb * dxb + dyb * dyb + dzb * dzb
            dnb = jnp.minimum(dist_ref[r, sl], db)
            dist_ref[r, sl] = dnb
            better = dnb > macc
            macc = jnp.maximum(macc, dnb)
            bacc = jnp.where(better, blk, bacc)
            xacc = jnp.where(better, pxb, xacc)
            yacc = jnp.where(better, pyb, yacc)
            zacc = jnp.where(better, pzb, zacc)
        iacc = bacc * _BLK + lane_blk
        m = jnp.max(macc, axis=1, keepdims=True)
        nxt = jnp.min(jnp.where(macc == m, iacc, big), axis=1, keepdims=True)
        win = iacc == nxt
        ncx = jnp.max(jnp.where(win, xacc, neg_inf), axis=1, keepdims=True)
        ncy = jnp.max(jnp.where(win, yacc, neg_inf), axis=1, keepdims=True)
        ncz = jnp.max(jnp.where(win, zacc, neg_inf), axis=1, keepdims=True)

        pos = jnp.bitwise_and(i, _FLUSH - 1)
        hit = slane == pos
        sfid_ref[r, :] = jnp.where(hit, nxt + boffs[h], sfid_ref[r, :])
        sx_ref[r, :] = jnp.where(hit, ncx, sx_ref[r, :])
        sy_ref[r, :] = jnp.where(hit, ncy, sy_ref[r, :])
        sz_ref[r, :] = jnp.where(hit, ncz, sz_ref[r, :])
        return ncx, ncy, ncz

    def body(i, carry):
        cxa, cya, cza, cxb, cyb, czb = carry
        ncxa, ncya, ncza = halfstep(0, i, cxa, cya, cza)
        ncxb, ncyb, nczb = halfstep(1, i, cxb, cyb, czb)
        pos = jnp.bitwise_and(i, _FLUSH - 1)

        @pl.when(pos == _FLUSH - 1)
        def _():
            base = pl.multiple_of(((i - s0) // _FLUSH) * _FLUSH, _FLUSH)
            osl = pl.ds(base, _FLUSH)
            fid_ref[:, osl] = sfid_ref[...]
            pox_ref[:, osl] = sx_ref[...]
            poy_ref[:, osl] = sy_ref[...]
            poz_ref[:, osl] = sz_ref[...]

        return ncxa, ncya, ncza, ncxb, ncyb, nczb

    fc = lax.fori_loop(max(1, s0), s0 + nsteps, body, tuple(carry0))
    for h in range(2):
        r = rows[h]
        cout_ref[r, 0:1] = fc[3 * h]
        cout_ref[r, 1:2] = fc[3 * h + 1]
        cout_ref[r, 2:3] = fc[3 * h + 2]


def _fps_chunk(px, py, pz, cin, din, s0, nsteps):
    out_i = jax.ShapeDtypeStruct((B, nsteps), jnp.int32)
    out_f = jax.ShapeDtypeStruct((B, nsteps), jnp.float32)
    out_c = jax.ShapeDtypeStruct((B, 8), jnp.float32)
    out_d = jax.ShapeDtypeStruct((B, N), jnp.float32)
    return pl.pallas_call(
        functools.partial(_fps_body, s0, nsteps),
        out_shape=(out_i, out_f, out_f, out_f, out_c, out_d),
        scratch_shapes=[
            pltpu.VMEM((B, _FLUSH), jnp.int32),
            pltpu.VMEM((B, _FLUSH), jnp.float32),
            pltpu.VMEM((B, _FLUSH), jnp.float32),
            pltpu.VMEM((B, _FLUSH), jnp.float32),
        ],
    )(px, py, pz, cin, din)


def _gather_max(h, sid, fid_flat):
    info = plsc.get_sparse_core_info()
    NC, NS = info.num_cores, info.num_subcores
    NW = NC * NS
    S = fid_flat.shape[0]
    PW = S // NW
    CP = 8
    NCHUNK = PW // CP
    ROWS = CP * K

    mesh = plsc.VectorSubcoreMesh(core_axis_name="c", subcore_axis_name="s")

    @functools.partial(
        pl.kernel,
        out_type=jax.ShapeDtypeStruct((S, COUT), jnp.float32),
        mesh=mesh,
        compiler_params=pltpu.CompilerParams(use_tc_tiling_on_sc=False),
        scratch_types=[
            pltpu.VMEM((PW,), jnp.int32),
            pltpu.VMEM((PW, K), jnp.int32),
            pltpu.VMEM((NCHUNK, ROWS), jnp.int32),
            pltpu.VMEM((ROWS, COUT), jnp.float32),
            pltpu.VMEM((ROWS, COUT), jnp.float32),
            pltpu.VMEM((PW, COUT), jnp.float32),
            pltpu.SemaphoreType.DMA,
            pltpu.SemaphoreType.DMA,
            pltpu.SemaphoreType.DMA,
        ],
    )
    def body(h_hbm, sid_hbm, fid_hbm, out_hbm, fid_v, nbr_v, nbrT, rows_a,
             rows_b, out_v, sem_s, sem_a, sem_b):
        wid = lax.axis_index("s") * NC + lax.axis_index("c")
        base = wid * PW
        pltpu.sync_copy(fid_hbm.at[pl.ds(base, PW)], fid_v)

        for q in range(PW // 128):
            pltpu.make_async_copy(
                sid_hbm.at[fid_v.at[pl.ds(q * 128, 128)]],
                nbr_v.at[pl.ds(q * 128, 128)], sem_s).start()
        for q in range(PW // 128):
            pltpu.make_async_copy(
                sid_hbm.at[fid_v.at[pl.ds(q * 128, 128)]],
                nbr_v.at[pl.ds(q * 128, 128)], sem_s).wait()

        def repack(c, _):
            for j in range(CP):
                nbrT[c, pl.ds(j * K, K)] = nbr_v[c * CP + j, :]
            return 0

        lax.fori_loop(0, NCHUNK, repack, 0)

        def h_copy(c, rows_buf, sem):
            return pltpu.make_async_copy(
                h_hbm.at[nbrT.at[c]], rows_buf, sem)

        def compute(c, rows_buf):
            def point(p, _):
                r0 = p * K
                for gr in range(COUT // 16):
                    sl = pl.ds(gr * 16, 16)
                    acc = rows_buf[r0, sl]
                    for k in range(1, K):
                        acc = jnp.maximum(acc, rows_buf[r0 + k, sl])
                    out_v[c * CP + p, sl] = acc
                return 0

            lax.fori_loop(0, CP, point, 0, unroll=2)

        h_copy(0, rows_a, sem_a).start()

        def step(i, _):
            ca = 2 * i
            cb = 2 * i + 1
            h_copy(cb, rows_b, sem_b).start()
            h_copy(ca, rows_a, sem_a).wait()
            compute(ca, rows_a)

            @pl.when(cb + 1 < NCHUNK)
            def _():
                h_copy(cb + 1, rows_a, sem_a).start()

            h_copy(cb, rows_b, sem_b).wait()
            compute(cb, rows_b)
            return 0

        lax.fori_loop(0, NCHUNK // 2, step, 0)
        pltpu.sync_copy(out_v, out_hbm.at[pl.ds(base, PW)])

    return body(h, sid, fid_flat)


def kernel(x, p, sid_euc, tid_euc, W, b, gamma, beta, running_mean,
           running_var):
    h = _mlp(x.reshape(B * N, CIN), W, b, gamma, beta, running_mean,
             running_var)
    px = p[:, :, 0]
    py = p[:, :, 1]
    pz = p[:, :, 2]
    CH = 4
    CS = NSAMP // CH
    cin = jnp.zeros((B, 8), jnp.float32)
    din = jnp.zeros((B, N), jnp.float32)
    xs, poxs, poys, pozs = [], [], [], []
    for c in range(CH):
        fid_c, pox_c, poy_c, poz_c, cin, din = _fps_chunk(
            px, py, pz, cin, din, c * CS, CS)
        xs.append(_gather_max(h, sid_euc, fid_c.reshape(-1))
                  .reshape(B, CS, COUT))
        poxs.append(pox_c)
        poys.append(poy_c)
        pozs.append(poz_c)
    x_out = jnp.concatenate(xs, axis=1)
    p_out = jnp.stack([
        jnp.concatenate(poxs, axis=1),
        jnp.concatenate(poys, axis=1),
        jnp.concatenate(pozs, axis=1),
    ], axis=-1)
    return x_out, p_out


# speedup vs baseline: 1.0217x; 1.0217x over previous
import functools
import math

import jax
import jax.numpy as jnp
from jax import lax
from jax.experimental import pallas as pl
from jax.experimental.pallas import tpu as pltpu
from jax.experimental.pallas import tpu_sc as plsc

B, N, CIN, COUT, K = 16, 4096, 128, 128, 16
NSAMP = 1024
BN_EPS = 1e-5


def _mlp_body(x_ref, w_ref, b_ref, gamma_ref, beta_ref, mean_ref, var_ref,
              o_ref):
    acc = jnp.dot(x_ref[...], w_ref[...], preferred_element_type=jnp.float32)
    scale = gamma_ref[...] / jnp.sqrt(var_ref[...] + BN_EPS)
    o_ref[...] = jnp.maximum(
        (acc + b_ref[...] - mean_ref[...]) * scale + beta_ref[...], 0.0)


def _mlp(xf, W, b, gamma, beta, mean, var):
    M = B * N
    BM = 2048
    vec = lambda v: v.reshape(1, COUT)
    return pl.pallas_call(
        _mlp_body,
        grid=(M // BM,),
        in_specs=[
            pl.BlockSpec((BM, CIN), lambda i: (i, 0)),
            pl.BlockSpec((CIN, COUT), lambda i: (0, 0)),
            pl.BlockSpec((1, COUT), lambda i: (0, 0)),
            pl.BlockSpec((1, COUT), lambda i: (0, 0)),
            pl.BlockSpec((1, COUT), lambda i: (0, 0)),
            pl.BlockSpec((1, COUT), lambda i: (0, 0)),
            pl.BlockSpec((1, COUT), lambda i: (0, 0)),
        ],
        out_specs=pl.BlockSpec((BM, COUT), lambda i: (i, 0)),
        out_shape=jax.ShapeDtypeStruct((M, COUT), jnp.float32),
    )(xf, W, vec(b), vec(gamma), vec(beta), vec(mean), vec(var))


_BLK = 128
_NBLK = N // _BLK
_FLUSH = 128


_HB = B // 2


def _fps_body(px_ref, py_ref, pz_ref, fid_ref, pox_ref, poy_ref,
              poz_ref, dist_ref, sfid_ref, sx_ref, sy_ref, sz_ref):
    lane_blk = lax.broadcasted_iota(jnp.int32, (_HB, _BLK), 1)
    slane = lax.broadcasted_iota(jnp.int32, (_HB, _FLUSH), 1)
    neg_inf = jnp.float32(-jnp.inf)
    big = jnp.int32(N)
    boffs = [
        (lax.broadcasted_iota(jnp.int32, (_HB, 1), 0) + h * _HB) * N
        for h in range(2)
    ]
    rows = [pl.ds(h * _HB, _HB) for h in range(2)]

    carry0 = []
    for h in range(2):
        r = rows[h]
        cx = px_ref[r, 0:1]
        cy = py_ref[r, 0:1]
        cz = pz_ref[r, 0:1]
        dx = px_ref[r, :] - cx
        dy = py_ref[r, :] - cy
        dz = pz_ref[r, :] - cz
        dist_ref[r, :] = dx * dx + dy * dy + dz * dz
        sfid_ref[r, :] = jnp.where(slane == 0, boffs[h], 0)
        sx_ref[r, :] = jnp.where(slane == 0, cx, 0.0)
        sy_ref[r, :] = jnp.where(slane == 0, cy, 0.0)
        sz_ref[r, :] = jnp.where(slane == 0, cz, 0.0)
        carry0 += [cx, cy, cz]

    def halfstep(h, i, cx, cy, cz):
        r = rows[h]
        macc = jnp.full((_HB, _BLK), neg_inf, jnp.float32)
        bacc = jnp.zeros((_HB, _BLK), jnp.int32)
        xacc = jnp.zeros((_HB, _BLK), jnp.float32)
        yacc = jnp.zeros((_HB, _BLK), jnp.float32)
        zacc = jnp.zeros((_HB, _BLK), jnp.float32)
        for blk in range(_NBLK):
            sl = pl.ds(blk * _BLK, _BLK)
            pxb = px_ref[r, sl]
            pyb = py_ref[r, sl]
            pzb = pz_ref[r, sl]
            dxb = pxb - cx
            dyb = pyb - cy
            dzb = pzb - cz
            db = dxb * dxb + dyb * dyb + dzb * dzb
            dnb = jnp.minimum(dist_ref[r, sl], db)
            dist_ref[r, sl] = dnb
            better = dnb > macc
            macc = jnp.maximum(macc, dnb)
            bacc = jnp.where(better, blk, bacc)
            xacc = jnp.where(better, pxb, xacc)
            yacc = jnp.where(better, pyb, yacc)
            zacc = jnp.where(better, pzb, zacc)
        iacc = bacc * _BLK + lane_blk
        m = jnp.max(macc, axis=1, keepdims=True)
        nxt = jnp.min(jnp.where(macc == m, iacc, big), axis=1, keepdims=True)
        win = iacc == nxt
        ncx = jnp.max(jnp.where(win, xacc, neg_inf), axis=1, keepdims=True)
        ncy = jnp.max(jnp.where(win, yacc, neg_inf), axis=1, keepdims=True)
        ncz = jnp.max(jnp.where(win, zacc, neg_inf), axis=1, keepdims=True)

        pos = jnp.bitwise_and(i, _FLUSH - 1)
        hit = slane == pos
        sfid_ref[r, :] = jnp.where(hit, nxt + boffs[h], sfid_ref[r, :])
        sx_ref[r, :] = jnp.where(hit, ncx, sx_ref[r, :])
        sy_ref[r, :] = jnp.where(hit, ncy, sy_ref[r, :])
        sz_ref[r, :] = jnp.where(hit, ncz, sz_ref[r, :])
        return ncx, ncy, ncz

    def body(i, carry):
        cxa, cya, cza, cxb, cyb, czb = carry
        ncxa, ncya, ncza = halfstep(0, i, cxa, cya, cza)
        ncxb, ncyb, nczb = halfstep(1, i, cxb, cyb, czb)
        pos = jnp.bitwise_and(i, _FLUSH - 1)

        @pl.when(pos == _FLUSH - 1)
        def _():
            base = pl.multiple_of((i // _FLUSH) * _FLUSH, _FLUSH)
            osl = pl.ds(base, _FLUSH)
            fid_ref[:, osl] = sfid_ref[...]
            pox_ref[:, osl] = sx_ref[...]
            poy_ref[:, osl] = sy_ref[...]
            poz_ref[:, osl] = sz_ref[...]

        return ncxa, ncya, ncza, ncxb, ncyb, nczb

    lax.fori_loop(1, NSAMP, body, tuple(carry0))


def _fps(px, py, pz):
    out_i = jax.ShapeDtypeStruct((B, NSAMP), jnp.int32)
    out_f = jax.ShapeDtypeStruct((B, NSAMP), jnp.float32)
    return pl.pallas_call(
        _fps_body,
        out_shape=(out_i, out_f, out_f, out_f),
        scratch_shapes=[
            pltpu.VMEM((B, N), jnp.float32),
            pltpu.VMEM((B, _FLUSH), jnp.int32),
            pltpu.VMEM((B, _FLUSH), jnp.float32),
            pltpu.VMEM((B, _FLUSH), jnp.float32),
            pltpu.VMEM((B, _FLUSH), jnp.float32),
        ],
    )(px, py, pz)


def _gather_max(h, sid, fid_flat):
    info = plsc.get_sparse_core_info()
    NC, NS = info.num_cores, info.num_subcores
    NW = NC * NS
    S = B * NSAMP
    PW = S // NW
    CP = 8
    NCHUNK = PW // CP
    ROWS = CP * K

    mesh = plsc.VectorSubcoreMesh(core_axis_name="c", subcore_axis_name="s")

    @functools.partial(
        pl.kernel,
        out_type=jax.ShapeDtypeStruct((S, COUT), jnp.float32),
        mesh=mesh,
        compiler_params=pltpu.CompilerParams(use_tc_tiling_on_sc=False),
        scratch_types=[
            pltpu.VMEM((PW,), jnp.int32),
            pltpu.VMEM((PW, K), jnp.int32),
            pltpu.VMEM((NCHUNK, ROWS), jnp.int32),
            pltpu.VMEM((ROWS, COUT), jnp.float32),
            pltpu.VMEM((ROWS, COUT), jnp.float32),
            pltpu.VMEM((PW, COUT), jnp.float32),
            pltpu.SemaphoreType.DMA,
            pltpu.SemaphoreType.DMA,
            pltpu.SemaphoreType.DMA,
        ],
    )
    def body(h_hbm, sid_hbm, fid_hbm, out_hbm, fid_v, nbr_v, nbrT, rows_a,
             rows_b, out_v, sem_s, sem_a, sem_b):
        wid = lax.axis_index("s") * NC + lax.axis_index("c")
        base = wid * PW
        pltpu.sync_copy(fid_hbm.at[pl.ds(base, PW)], fid_v)

        for q in range(PW // 128):
            pltpu.make_async_copy(
                sid_hbm.at[fid_v.at[pl.ds(q * 128, 128)]],
                nbr_v.at[pl.ds(q * 128, 128)], sem_s).start()
        for q in range(PW // 128):
            pltpu.make_async_copy(
                sid_hbm.at[fid_v.at[pl.ds(q * 128, 128)]],
                nbr_v.at[pl.ds(q * 128, 128)], sem_s).wait()

        def repack(c, _):
            for j in range(CP):
                nbrT[c, pl.ds(j * K, K)] = nbr_v[c * CP + j, :]
            return 0

        lax.fori_loop(0, NCHUNK, repack, 0)

        def h_copy(c, rows_buf, sem):
            return pltpu.make_async_copy(
                h_hbm.at[nbrT.at[c]], rows_buf, sem)

        def compute(c, rows_buf):
            def point(p, _):
                r0 = p * K
                for gr in range(COUT // 16):
                    sl = pl.ds(gr * 16, 16)
                    acc = rows_buf[r0, sl]
                    for k in range(1, K):
                        acc = jnp.maximum(acc, rows_buf[r0 + k, sl])
                    out_v[c * CP + p, sl] = acc
                return 0

            lax.fori_loop(0, CP, point, 0, unroll=2)

        h_copy(0, rows_a, sem_a).start()

        def step(i, _):
            ca = 2 * i
            cb = 2 * i + 1
            h_copy(cb, rows_b, sem_b).start()
            h_copy(ca, rows_a, sem_a).wait()
            compute(ca, rows_a)

            @pl.when(cb + 1 < NCHUNK)
            def _():
                h_copy(cb + 1, rows_a, sem_a).start()

            h_copy(cb, rows_b, sem_b).wait()
            compute(cb, rows_b)
            return 0

        lax.fori_loop(0, NCHUNK // 2, step, 0)
        pltpu.sync_copy(out_v, out_hbm.at[pl.ds(base, PW)])

    return body(h, sid, fid_flat)


def kernel(x, p, sid_euc, tid_euc, W, b, gamma, beta, running_mean,
           running_var):
    h = _mlp(x.reshape(B * N, CIN), W, b, gamma, beta, running_mean,
             running_var)
    px = p[:, :, 0]
    py = p[:, :, 1]
    pz = p[:, :, 2]
    fid, pox, poy, poz = _fps(px, py, pz)
    x_out = _gather_max(h, sid_euc, fid.reshape(-1))
    p_out = jnp.stack([pox, poy, poz], axis=-1)
    return x_out.reshape(B, NSAMP, COUT), p_out


# speedup vs baseline: 1.1463x; 1.1219x over previous
import functools
import math

import jax
import jax.numpy as jnp
from jax import lax
from jax.experimental import pallas as pl
from jax.experimental.pallas import tpu as pltpu
from jax.experimental.pallas import tpu_sc as plsc

B, N, CIN, COUT, K = 16, 4096, 128, 128, 16
NSAMP = 1024
BN_EPS = 1e-5


def _mlp_body(x_ref, w_ref, b_ref, gamma_ref, beta_ref, mean_ref, var_ref,
              o_ref):
    acc = jnp.dot(x_ref[...], w_ref[...], preferred_element_type=jnp.float32)
    scale = gamma_ref[...] / jnp.sqrt(var_ref[...] + BN_EPS)
    o_ref[...] = jnp.maximum(
        (acc + b_ref[...] - mean_ref[...]) * scale + beta_ref[...], 0.0)


def _mlp(xf, W, b, gamma, beta, mean, var):
    M = B * N
    BM = 2048
    vec = lambda v: v.reshape(1, COUT)
    return pl.pallas_call(
        _mlp_body,
        grid=(M // BM,),
        in_specs=[
            pl.BlockSpec((BM, CIN), lambda i: (i, 0)),
            pl.BlockSpec((CIN, COUT), lambda i: (0, 0)),
            pl.BlockSpec((1, COUT), lambda i: (0, 0)),
            pl.BlockSpec((1, COUT), lambda i: (0, 0)),
            pl.BlockSpec((1, COUT), lambda i: (0, 0)),
            pl.BlockSpec((1, COUT), lambda i: (0, 0)),
            pl.BlockSpec((1, COUT), lambda i: (0, 0)),
        ],
        out_specs=pl.BlockSpec((BM, COUT), lambda i: (i, 0)),
        out_shape=jax.ShapeDtypeStruct((M, COUT), jnp.float32),
    )(xf, W, vec(b), vec(gamma), vec(beta), vec(mean), vec(var))


_BLK = 128
_NBLK = N // _BLK
_FLUSH = 128


_HB = B // 2


def _fps_body(px_ref, py_ref, pz_ref, fid_ref, pox_ref, poy_ref,
              poz_ref, dist_ref, sfid_ref, sx_ref, sy_ref, sz_ref):
    lane_blk = lax.broadcasted_iota(jnp.int32, (_HB, _BLK), 1)
    slane = lax.broadcasted_iota(jnp.int32, (_HB, _FLUSH), 1)
    neg_inf = jnp.float32(-jnp.inf)
    big = jnp.int32(N)
    boffs = [
        (lax.broadcasted_iota(jnp.int32, (_HB, 1), 0) + h * _HB) * N
        for h in range(2)
    ]
    rows = [pl.ds(h * _HB, _HB) for h in range(2)]

    carry0 = []
    for h in range(2):
        r = rows[h]
        cx = px_ref[r, 0:1]
        cy = py_ref[r, 0:1]
        cz = pz_ref[r, 0:1]
        dx = px_ref[r, :] - cx
        dy = py_ref[r, :] - cy
        dz = pz_ref[r, :] - cz
        dist_ref[r, :] = dx * dx + dy * dy + dz * dz
        sfid_ref[r, :] = jnp.where(slane == 0, boffs[h], 0)
        sx_ref[r, :] = jnp.where(slane == 0, cx, 0.0)
        sy_ref[r, :] = jnp.where(slane == 0, cy, 0.0)
        sz_ref[r, :] = jnp.where(slane == 0, cz, 0.0)
        carry0 += [jnp.broadcast_to(cx, (_HB, _BLK)),
                   jnp.broadcast_to(cy, (_HB, _BLK)),
                   jnp.broadcast_to(cz, (_HB, _BLK))]

    def halfstep(h, i, cx, cy, cz):
        r = rows[h]
        macc = jnp.full((_HB, _BLK), neg_inf, jnp.float32)
        bacc = jnp.zeros((_HB, _BLK), jnp.int32)
        xacc = jnp.zeros((_HB, _BLK), jnp.float32)
        yacc = jnp.zeros((_HB, _BLK), jnp.float32)
        zacc = jnp.zeros((_HB, _BLK), jnp.float32)
        for blk in range(_NBLK):
            sl = pl.ds(blk * _BLK, _BLK)
            pxb = px_ref[r, sl]
            pyb = py_ref[r, sl]
            pzb = pz_ref[r, sl]
            dxb = pxb - cx
            dyb = pyb - cy
            dzb = pzb - cz
            db = dxb * dxb + dyb * dyb + dzb * dzb
            dnb = jnp.minimum(dist_ref[r, sl], db)
            dist_ref[r, sl] = dnb
            better = dnb > macc
            macc = jnp.maximum(macc, dnb)
            bacc = jnp.where(better, blk, bacc)
            xacc = jnp.where(better, pxb, xacc)
            yacc = jnp.where(better, pyb, yacc)
            zacc = jnp.where(better, pzb, zacc)
        iacc = bacc * _BLK + lane_blk
        m = jnp.max(macc, axis=1, keepdims=True)
        nxt = jnp.min(jnp.where(macc == m, iacc, big), axis=1, keepdims=True)
        win = iacc == nxt
        ncx = jnp.max(jnp.where(win, xacc, neg_inf), axis=1, keepdims=True)
        ncy = jnp.max(jnp.where(win, yacc, neg_inf), axis=1, keepdims=True)
        ncz = jnp.max(jnp.where(win, zacc, neg_inf), axis=1, keepdims=True)

        pos = jnp.bitwise_and(i, _FLUSH - 1)
        hit = slane == pos
        sfid_ref[r, :] = jnp.where(hit, nxt + boffs[h], sfid_ref[r, :])
        sx_ref[r, :] = jnp.where(hit, ncx, sx_ref[r, :])
        sy_ref[r, :] = jnp.where(hit, ncy, sy_ref[r, :])
        sz_ref[r, :] = jnp.where(hit, ncz, sz_ref[r, :])
        return (jnp.broadcast_to(ncx, (_HB, _BLK)),
                jnp.broadcast_to(ncy, (_HB, _BLK)),
                jnp.broadcast_to(ncz, (_HB, _BLK)))

    def body(i, carry):
        cxa, cya, cza, cxb, cyb, czb = carry
        ncxa, ncya, ncza = halfstep(0, i, cxa, cya, cza)
        ncxb, ncyb, nczb = halfstep(1, i, cxb, cyb, czb)
        pos = jnp.bitwise_and(i, _FLUSH - 1)

        @pl.when(pos == _FLUSH - 1)
        def _():
            base = pl.multiple_of((i // _FLUSH) * _FLUSH, _FLUSH)
            osl = pl.ds(base, _FLUSH)
            fid_ref[:, osl] = sfid_ref[...]
            pox_ref[:, osl] = sx_ref[...]
            poy_ref[:, osl] = sy_ref[...]
            poz_ref[:, osl] = sz_ref[...]

        return ncxa, ncya, ncza, ncxb, ncyb, nczb

    lax.fori_loop(1, NSAMP, body, tuple(carry0))


def _fps(px, py, pz):
    out_i = jax.ShapeDtypeStruct((B, NSAMP), jnp.int32)
    out_f = jax.ShapeDtypeStruct((B, NSAMP), jnp.float32)
    return pl.pallas_call(
        _fps_body,
        out_shape=(out_i, out_f, out_f, out_f),
        scratch_shapes=[
            pltpu.VMEM((B, N), jnp.float32),
            pltpu.VMEM((B, _FLUSH), jnp.int32),
            pltpu.VMEM((B, _FLUSH), jnp.float32),
            pltpu.VMEM((B, _FLUSH), jnp.float32),
            pltpu.VMEM((B, _FLUSH), jnp.float32),
        ],
    )(px, py, pz)


def _gather_max(h, sid, fid_flat):
    info = plsc.get_sparse_core_info()
    NC, NS = info.num_cores, info.num_subcores
    NW = NC * NS
    S = B * NSAMP
    PW = S // NW
    CP = 8
    NCHUNK = PW // CP
    ROWS = CP * K

    mesh = plsc.VectorSubcoreMesh(core_axis_name="c", subcore_axis_name="s")

    @functools.partial(
        pl.kernel,
        out_type=jax.ShapeDtypeStruct((S, COUT), jnp.float32),
        mesh=mesh,
        compiler_params=pltpu.CompilerParams(use_tc_tiling_on_sc=False),
        scratch_types=[
            pltpu.VMEM((PW,), jnp.int32),
            pltpu.VMEM((PW, K), jnp.int32),
            pltpu.VMEM((NCHUNK, ROWS), jnp.int32),
            pltpu.VMEM((ROWS, COUT), jnp.float32),
            pltpu.VMEM((ROWS, COUT), jnp.float32),
            pltpu.VMEM((PW, COUT), jnp.float32),
            pltpu.SemaphoreType.DMA,
            pltpu.SemaphoreType.DMA,
            pltpu.SemaphoreType.DMA,
        ],
    )
    def body(h_hbm, sid_hbm, fid_hbm, out_hbm, fid_v, nbr_v, nbrT, rows_a,
             rows_b, out_v, sem_s, sem_a, sem_b):
        wid = lax.axis_index("s") * NC + lax.axis_index("c")
        base = wid * PW
        pltpu.sync_copy(fid_hbm.at[pl.ds(base, PW)], fid_v)

        for q in range(PW // 128):
            pltpu.make_async_copy(
                sid_hbm.at[fid_v.at[pl.ds(q * 128, 128)]],
                nbr_v.at[pl.ds(q * 128, 128)], sem_s).start()
        for q in range(PW // 128):
            pltpu.make_async_copy(
                sid_hbm.at[fid_v.at[pl.ds(q * 128, 128)]],
                nbr_v.at[pl.ds(q * 128, 128)], sem_s).wait()

        def repack(c, _):
            for j in range(CP):
                nbrT[c, pl.ds(j * K, K)] = nbr_v[c * CP + j, :]
            return 0

        lax.fori_loop(0, NCHUNK, repack, 0)

        def h_copy(c, rows_buf, sem):
            return pltpu.make_async_copy(
                h_hbm.at[nbrT.at[c]], rows_buf, sem)

        def compute(c, rows_buf):
            def point(p, _):
                r0 = p * K
                for gr in range(COUT // 16):
                    sl = pl.ds(gr * 16, 16)
                    acc = rows_buf[r0, sl]
                    for k in range(1, K):
                        acc = jnp.maximum(acc, rows_buf[r0 + k, sl])
                    out_v[c * CP + p, sl] = acc
                return 0

            lax.fori_loop(0, CP, point, 0, unroll=2)

        h_copy(0, rows_a, sem_a).start()

        def step(i, _):
            ca = 2 * i
            cb = 2 * i + 1
            h_copy(cb, rows_b, sem_b).start()
            h_copy(ca, rows_a, sem_a).wait()
            compute(ca, rows_a)

            @pl.when(cb + 1 < NCHUNK)
            def _():
                h_copy(cb + 1, rows_a, sem_a).start()

            h_copy(cb, rows_b, sem_b).wait()
            compute(cb, rows_b)
            return 0

        lax.fori_loop(0, NCHUNK // 2, step, 0)
        pltpu.sync_copy(out_v, out_hbm.at[pl.ds(base, PW)])

    return body(h, sid, fid_flat)


def kernel(x, p, sid_euc, tid_euc, W, b, gamma, beta, running_mean,
           running_var):
    h = _mlp(x.reshape(B * N, CIN), W, b, gamma, beta, running_mean,
             running_var)
    px = p[:, :, 0]
    py = p[:, :, 1]
    pz = p[:, :, 2]
    fid, pox, poy, poz = _fps(px, py, pz)
    x_out = _gather_max(h, sid_euc, fid.reshape(-1))
    p_out = jnp.stack([pox, poy, poz], axis=-1)
    return x_out.reshape(B, NSAMP, COUT), p_out
